# baseline (speedup 1.0000x reference)
import jax
import jax.numpy as jnp
from jax.experimental import pallas as pl
from jax.experimental.pallas import tpu as pltpu

_IN = 10
_OUT = 5
_TBT = 2048


def _linear_tiles_kernel(x_ref, wt_ref, b_ref, o_ref):
    t = x_ref.shape[0]
    x2 = x_ref[...].reshape(t * 8, _IN)
    y = jnp.dot(x2, wt_ref[...], preferred_element_type=jnp.float32)
    o_ref[...] = (y + b_ref[...]).reshape(t, 8, _OUT).astype(o_ref.dtype)


def _linear_chunk(xc, wt, b2, T):
    cost = pl.CostEstimate(
        flops=2 * T * 8 * _IN * _OUT,
        transcendentals=0,
        bytes_accessed=T * 2 * 8 * 128 * 4,
    )
    return pl.pallas_call(
        _linear_tiles_kernel,
        out_shape=jax.ShapeDtypeStruct((T, 8, _OUT), xc.dtype),
        grid=(pl.cdiv(T, _TBT),),
        in_specs=[
            pl.BlockSpec((_TBT, 8, _IN), lambda i: (i, 0, 0)),
            pl.BlockSpec((_IN, _OUT), lambda i: (0, 0)),
            pl.BlockSpec((1, _OUT), lambda i: (0, 0)),
        ],
        out_specs=pl.BlockSpec((_TBT, 8, _OUT), lambda i: (i, 0, 0)),
        cost_estimate=cost,
        compiler_params=pltpu.CompilerParams(
            dimension_semantics=("parallel",),
        ),
    )(xc, wt, b2)


@jax.jit
def _forward(x, w, b):
    B = x.shape[0]
    Bp = ((B + 7) // 8) * 8
    if Bp != B:
        x = jnp.pad(x, ((0, Bp - B), (0, 0)))
    T = Bp // 8

    wt = w.T.astype(x.dtype)
    b2 = b.reshape(1, _OUT).astype(x.dtype)

    NC = 4
    if T % (NC * _TBT) != 0:
        NC = 1
    Tc = T // NC
    Bc = Bp // NC
    outs = []
    for c in range(NC):
        xc = x[c * Bc:(c + 1) * Bc].reshape(Tc, 8, _IN)
        outs.append(_linear_chunk(xc, wt, b2, Tc).reshape(Bc, _OUT))
    out = outs[0] if NC == 1 else jnp.concatenate(outs, axis=0)
    return out[:B]


def kernel(x, w, b):
    return _forward(x, w, b)


# speedup vs baseline: 1.2541x; 1.2541x over previous
import jax
import jax.numpy as jnp
from jax.experimental import pallas as pl
from jax.experimental.pallas import tpu as pltpu

_IN = 10
_OUT = 5
_TBT = 2048


def _linear_tiles_kernel(x_ref, wt_ref, b_ref, o_ref):
    t = o_ref.shape[0]
    y = jnp.dot(x_ref[...], wt_ref[...], preferred_element_type=jnp.float32)
    o_ref[...] = (y + b_ref[...]).reshape(t, 8, _OUT).astype(o_ref.dtype)


@jax.jit
def _forward(x, w, b):
    B = x.shape[0]
    Bp = ((B + 7) // 8) * 8
    if Bp != B:
        x = jnp.pad(x, ((0, Bp - B), (0, 0)))
    T = Bp // 8

    wt = w.T.astype(x.dtype)
    b2 = b.reshape(1, _OUT).astype(x.dtype)

    cost = pl.CostEstimate(
        flops=2 * Bp * _IN * _OUT,
        transcendentals=0,
        bytes_accessed=T * 2 * 8 * 128 * 4,
    )

    out = pl.pallas_call(
        _linear_tiles_kernel,
        out_shape=jax.ShapeDtypeStruct((T, 8, _OUT), x.dtype),
        grid=(pl.cdiv(T, _TBT),),
        in_specs=[
            pl.BlockSpec((_TBT * 8, _IN), lambda i: (i, 0)),
            pl.BlockSpec((_IN, _OUT), lambda i: (0, 0)),
            pl.BlockSpec((1, _OUT), lambda i: (0, 0)),
        ],
        out_specs=pl.BlockSpec((_TBT, 8, _OUT), lambda i: (i, 0, 0)),
        cost_estimate=cost,
        compiler_params=pltpu.CompilerParams(
            dimension_semantics=("parallel",),
        ),
    )(x, wt, b2)

    return out.reshape(Bp, _OUT)[:B]


def kernel(x, w, b):
    return _forward(x, w, b)


# speedup vs baseline: 1.6788x; 1.3387x over previous
import jax
import jax.numpy as jnp
from jax.experimental import pallas as pl
from jax.experimental.pallas import tpu as pltpu

_IN = 10
_OUT = 5
_TBT = 2048


def _linear_tiles_kernel(x_ref, wt_ref, b_ref, o_ref):
    t = o_ref.shape[0]
    x2 = x_ref[...].reshape(t * 8, _IN)
    y = jnp.dot(x2, wt_ref[...], preferred_element_type=jnp.float32)
    o_ref[...] = (y + b_ref[...]).reshape(t, 8, _OUT).astype(o_ref.dtype)


@jax.jit
def _forward(x, w, b):
    B = x.shape[0]
    Bp = ((B + 7) // 8) * 8
    if Bp != B:
        x = jnp.pad(x, ((0, Bp - B), (0, 0)))
    T = Bp // 8
    xv = x.reshape(T, 8, _IN)

    wt = w.T.astype(x.dtype)
    b2 = b.reshape(1, _OUT).astype(x.dtype)

    cost = pl.CostEstimate(
        flops=2 * Bp * _IN * _OUT,
        transcendentals=0,
        bytes_accessed=T * 2 * 8 * 128 * 4,
    )

    out = pl.pallas_call(
        _linear_tiles_kernel,
        out_shape=jax.ShapeDtypeStruct((T, 8, _OUT), x.dtype),
        grid=(pl.cdiv(T, _TBT),),
        in_specs=[
            pl.BlockSpec((_TBT, 8, _IN), lambda i: (i, 0, 0)),
            pl.BlockSpec((_IN, _OUT), lambda i: (0, 0)),
            pl.BlockSpec((1, _OUT), lambda i: (0, 0)),
        ],
        out_specs=pl.BlockSpec((_TBT, 8, _OUT), lambda i: (i, 0, 0)),
        cost_estimate=cost,
        compiler_params=pltpu.CompilerParams(
            dimension_semantics=("parallel",),
        ),
    )(xv, wt, b2)

    return out.reshape(Bp, _OUT)[:B]


def kernel(x, w, b):
    return _forward(x, w, b)


# speedup vs baseline: 1.6817x; 1.0017x over previous
import jax
import jax.numpy as jnp
from jax.experimental import pallas as pl
from jax.experimental.pallas import tpu as pltpu

_IN = 10
_OUT = 5
_TBT = 1024


def _linear_tiles_kernel(x_ref, wt_ref, b_ref, o_ref):
    t = o_ref.shape[0]
    x2 = x_ref[...].reshape(t * 16, _IN)
    y = jnp.dot(x2, wt_ref[...], preferred_element_type=jnp.float32)
    o_ref[...] = (y + b_ref[...]).reshape(t, 16, _OUT).astype(o_ref.dtype)


@jax.jit
def _forward(x, w, b):
    B = x.shape[0]
    Bp = ((B + 15) // 16) * 16
    if Bp != B:
        x = jnp.pad(x, ((0, Bp - B), (0, 0)))
    T = Bp // 16
    xv = x.reshape(T, 16, _IN)

    wt = w.T.astype(x.dtype)
    b2 = b.reshape(1, _OUT).astype(x.dtype)

    cost = pl.CostEstimate(
        flops=2 * Bp * _IN * _OUT,
        transcendentals=0,
        bytes_accessed=T * 2 * 16 * 128 * 4,
    )

    out = pl.pallas_call(
        _linear_tiles_kernel,
        out_shape=jax.ShapeDtypeStruct((T, 16, _OUT), x.dtype),
        grid=(pl.cdiv(T, _TBT),),
        in_specs=[
            pl.BlockSpec((_TBT, 16, _IN), lambda i: (i, 0, 0)),
            pl.BlockSpec((_IN, _OUT), lambda i: (0, 0)),
            pl.BlockSpec((1, _OUT), lambda i: (0, 0)),
        ],
        out_specs=pl.BlockSpec((_TBT, 16, _OUT), lambda i: (i, 0, 0)),
        cost_estimate=cost,
        compiler_params=pltpu.CompilerParams(
            dimension_semantics=("parallel",),
        ),
    )(xv, wt, b2)

    return out.reshape(Bp, _OUT)[:B]


def kernel(x, w, b):
    return _forward(x, w, b)
